# baseline (speedup 1.0000x reference)
import functools

import jax
import jax.numpy as jnp
from jax import lax
from jax.experimental import pallas as pl
from jax.experimental.pallas import tpu as pltpu
from jax.experimental.pallas import tpu_sc as plsc

N = 10000
E = 320000

NUM_CORES = 2
NUM_SUBCORES = 16
K_EDGES = 512
EDGES_PER_CORE = 163840
E_PAD = EDGES_PER_CORE * NUM_CORES
CHUNKS = EDGES_PER_CORE // K_EDGES
GROUPS = K_EDGES // 16


def _make_spmm(do: int):
    C = do // NUM_SUBCORES
    mesh = plsc.VectorSubcoreMesh(
        core_axis_name="c", subcore_axis_name="s",
        num_cores=NUM_CORES, num_subcores=NUM_SUBCORES)

    @functools.partial(
        pl.kernel,
        out_type=jax.ShapeDtypeStruct((NUM_CORES, do, N), jnp.float32),
        mesh=mesh,
        compiler_params=pltpu.CompilerParams(
            needs_layout_passes=False, use_tc_tiling_on_sc=False),
        scratch_types=[
            pltpu.VMEM((3, K_EDGES), jnp.int32),
            pltpu.VMEM((3, K_EDGES), jnp.int32),
            pltpu.VMEM((C, N), jnp.float32),
            pltpu.VMEM((C, N), jnp.float32),
            pltpu.SemaphoreType.DMA,
            pltpu.SemaphoreType.DMA,
        ],
    )
    def spmm(sup_hbm, edata_hbm, out_hbm, eb0, eb1, sup, acc, se0, se1):
        cid = lax.axis_index("c")
        sid = lax.axis_index("s")
        f0 = sid * C

        pltpu.sync_copy(sup_hbm.at[pl.ds(f0, C)], sup)
        zero16 = jnp.zeros((16,), jnp.float32)

        def zfill(i, carry):
            for c in range(C):
                acc[c, pl.ds(i * 16, 16)] = zero16
            return carry

        lax.fori_loop(0, N // 16, zfill, 0)

        eb = (eb0, eb1)
        se = (se0, se1)

        def work(b, j):
            @plsc.parallel_loop(0, GROUPS, 1, unroll=8)
            def group(g):
                src16 = eb[b][0, pl.ds(g * 16, 16)]
                dst16 = eb[b][1, pl.ds(g * 16, 16)]
                a16 = plsc.bitcast(eb[b][2, pl.ds(g * 16, 16)], jnp.float32)
                for c in range(C):
                    v = plsc.load_gather(sup.at[c], [src16])
                    plsc.addupdate_scatter(acc.at[c], [dst16], v * a16)

        pltpu.async_copy(edata_hbm.at[cid, 0], eb0, se0)

        def pair(t, carry):
            pltpu.make_async_copy(edata_hbm.at[cid, 0], eb0, se0).wait()
            pltpu.async_copy(edata_hbm.at[cid, 2 * t + 1], eb1, se1)
            work(0, 2 * t)
            pltpu.make_async_copy(edata_hbm.at[cid, 0], eb1, se1).wait()
            pltpu.async_copy(edata_hbm.at[cid, 2 * t + 2], eb0, se0)
            work(1, 2 * t + 1)
            return carry

        lax.fori_loop(0, CHUNKS // 2, pair, 0)
        pltpu.make_async_copy(edata_hbm.at[cid, 0], eb0, se0).wait()

        pltpu.sync_copy(acc, out_hbm.at[cid, pl.ds(f0, C)])

    return spmm


_spmm = {d: _make_spmm(d) for d in (16, 32, 64)}


def _mm0_body(xt_ref, wt_ref, o_ref):
    o_ref[...] = jnp.dot(wt_ref[...], xt_ref[...],
                         preferred_element_type=jnp.float32)


def _tc_mm0(xt, wt):
    return pl.pallas_call(
        _mm0_body,
        out_shape=jax.ShapeDtypeStruct((wt.shape[0], N), jnp.float32),
    )(xt, wt)


def _fused_body(b_ref, g_ref, beta_ref, wt_ref, o_ref, *p_refs):
    blocks = [p_refs[i][0] + p_refs[i][1] for i in range(len(p_refs))]
    s = blocks[0] if len(blocks) == 1 else jnp.concatenate(blocks, axis=0)
    s = s + b_ref[...]
    mu = jnp.mean(s, axis=1, keepdims=True)
    xc = s - mu
    var = jnp.mean(xc * xc, axis=1, keepdims=True)
    h = xc * lax.rsqrt(var + 1e-5) * g_ref[...] + beta_ref[...]
    h = jnp.maximum(h, 0.0)
    o_ref[...] = jnp.dot(wt_ref[...], h, preferred_element_type=jnp.float32)


def _tc_fused(parts, b, g, beta, wt):
    def wrapped(b_r, g_r, be_r, w_r, *ps_and_o):
        ps, o = ps_and_o[:-1], ps_and_o[-1]
        _fused_body(b_r, g_r, be_r, w_r, o, *ps)

    return pl.pallas_call(
        wrapped,
        out_shape=jax.ShapeDtypeStruct((wt.shape[0], N), jnp.float32),
    )(b.reshape(-1, 1), g.reshape(-1, 1), beta.reshape(-1, 1), wt, *parts)


def _final_body(p_ref, b_ref, zm_ref, zs_ref):
    q = p_ref[0] + p_ref[1] + b_ref[...]
    zm_ref[...] = q[:32, :].T
    zs_ref[...] = q[32:, :].T


def _tc_final(p, b67):
    return pl.pallas_call(
        _final_body,
        out_shape=(jax.ShapeDtypeStruct((N, 32), jnp.float32),
                   jax.ShapeDtypeStruct((N, 32), jnp.float32)),
    )(p, b67.reshape(-1, 1))


def kernel(x, edge_index, adj_values, W1, b1, W2, b2, W3, b3, W4, b4,
           W5, b5, W6, b6, W7, b7, g1, beta1, g2, beta2, g3, beta3,
           g4, beta4, g5, beta5):
    pad = E_PAD - E
    src = jnp.concatenate([edge_index[0], jnp.zeros((pad,), jnp.int32)])
    dst = jnp.concatenate([edge_index[1], jnp.zeros((pad,), jnp.int32)])
    adj = jnp.concatenate([adj_values, jnp.zeros((pad,), jnp.float32)])
    edata = jnp.stack(
        [src.reshape(NUM_CORES, CHUNKS, K_EDGES),
         dst.reshape(NUM_CORES, CHUNKS, K_EDGES),
         lax.bitcast_convert_type(adj, jnp.int32).reshape(
             NUM_CORES, CHUNKS, K_EDGES)], axis=2)
    edata = jnp.pad(edata, ((0, 0), (0, 1), (0, 0), (0, 0)))

    def spmm(sup_t):
        d = sup_t.shape[0]
        if d <= 64:
            p = _spmm[d](sup_t, edata)
            return [p]
        return [_spmm[64](sup_t[:64], edata), _spmm[64](sup_t[64:], edata)]

    sup = _tc_mm0(x.T, W1.T)
    parts = spmm(sup)
    sup = _tc_fused(parts, b1, g1, beta1, W2.T)
    parts = spmm(sup)
    sup = _tc_fused(parts, b2, g2, beta2, W3.T)
    parts = spmm(sup)
    sup = _tc_fused(parts, b3, g3, beta3, W4.T)
    parts = spmm(sup)
    sup = _tc_fused(parts, b4, g4, beta4, W5.T)
    parts = spmm(sup)
    W67t = jnp.concatenate([W6, W7], axis=1).T
    sup = _tc_fused(parts, b5, g5, beta5, W67t)
    parts = spmm(sup)
    b67 = jnp.concatenate([b6, b7])
    z_mean, z_std = _tc_final(parts[0], b67)
    return (z_mean, z_std)


# speedup vs baseline: 1.1873x; 1.1873x over previous
import functools

import jax
import jax.numpy as jnp
from jax import lax
from jax.experimental import pallas as pl
from jax.experimental.pallas import tpu as pltpu
from jax.experimental.pallas import tpu_sc as plsc

N = 10000
E = 320000

NUM_CORES = 2
NUM_SUBCORES = 16
K_EDGES = 512
EDGES_PER_CORE = 163840
E_PAD = EDGES_PER_CORE * NUM_CORES
CHUNKS = EDGES_PER_CORE // K_EDGES
GROUPS = K_EDGES // 16


def _make_spmm(do: int):
    C = do // NUM_SUBCORES
    mesh = plsc.VectorSubcoreMesh(
        core_axis_name="c", subcore_axis_name="s",
        num_cores=NUM_CORES, num_subcores=NUM_SUBCORES)

    @functools.partial(
        pl.kernel,
        out_type=jax.ShapeDtypeStruct((NUM_CORES, do, N), jnp.float32),
        mesh=mesh,
        compiler_params=pltpu.CompilerParams(
            needs_layout_passes=False, use_tc_tiling_on_sc=False),
        scratch_types=[
            pltpu.VMEM((3, K_EDGES), jnp.int32),
            pltpu.VMEM((3, K_EDGES), jnp.int32),
            pltpu.VMEM((C, N), jnp.float32),
            pltpu.VMEM((C, N), jnp.float32),
            pltpu.SemaphoreType.DMA,
            pltpu.SemaphoreType.DMA,
        ],
    )
    def spmm(sup_hbm, edata_hbm, out_hbm, eb0, eb1, sup, acc, se0, se1):
        cid = lax.axis_index("c")
        sid = lax.axis_index("s")
        f0 = sid * C

        pltpu.sync_copy(sup_hbm.at[pl.ds(f0, C)], sup)
        zero16 = jnp.zeros((16,), jnp.float32)

        def zfill(i, carry):
            for c in range(C):
                acc[c, pl.ds(i * 16, 16)] = zero16
            return carry

        lax.fori_loop(0, N // 16, zfill, 0)

        eb = (eb0, eb1)
        se = (se0, se1)

        def work(b, j):
            @plsc.parallel_loop(0, GROUPS, 1, unroll=8)
            def group(g):
                src16 = eb[b][0, pl.ds(g * 16, 16)]
                dst16 = eb[b][1, pl.ds(g * 16, 16)]
                a16 = plsc.bitcast(eb[b][2, pl.ds(g * 16, 16)], jnp.float32)
                for c in range(C):
                    v = plsc.load_gather(sup.at[c], [src16])
                    plsc.addupdate_scatter(acc.at[c], [dst16], v * a16)

        pltpu.async_copy(edata_hbm.at[cid, 0], eb0, se0)

        def pair(t, carry):
            pltpu.make_async_copy(edata_hbm.at[cid, 0], eb0, se0).wait()
            pltpu.async_copy(edata_hbm.at[cid, 2 * t + 1], eb1, se1)
            work(0, 2 * t)
            pltpu.make_async_copy(edata_hbm.at[cid, 0], eb1, se1).wait()
            pltpu.async_copy(edata_hbm.at[cid, 2 * t + 2], eb0, se0)
            work(1, 2 * t + 1)
            return carry

        lax.fori_loop(0, CHUNKS // 2, pair, 0)
        pltpu.make_async_copy(edata_hbm.at[cid, 0], eb0, se0).wait()

        pltpu.sync_copy(acc, out_hbm.at[cid, pl.ds(f0, C)])

    return spmm


_spmm = {d: _make_spmm(d) for d in (16, 32, 64)}


def _mm0_body(xt_ref, wt_ref, o_ref):
    o_ref[...] = jnp.dot(wt_ref[...], xt_ref[...],
                         preferred_element_type=jnp.float32)


def _tc_mm0(xt, wt):
    return pl.pallas_call(
        _mm0_body,
        out_shape=jax.ShapeDtypeStruct((wt.shape[0], N), jnp.float32),
    )(xt, wt)


def _bn_relu(s, g, beta):
    mu = jnp.mean(s, axis=1, keepdims=True)
    xc = s - mu
    var = jnp.mean(xc * xc, axis=1, keepdims=True)
    return jnp.maximum(xc * lax.rsqrt(var + 1e-5) * g + beta, 0.0)


def _sum_bn_body(b_ref, g_ref, beta_ref, p_ref, o_ref):
    s = p_ref[0] + p_ref[1] + b_ref[...]
    o_ref[...] = _bn_relu(s, g_ref[...], beta_ref[...])


def _tc_sum_bn(p, b, g, beta):
    return pl.pallas_call(
        _sum_bn_body,
        out_shape=jax.ShapeDtypeStruct((p.shape[1], N), jnp.float32),
    )(b.reshape(-1, 1), g.reshape(-1, 1), beta.reshape(-1, 1), p)


def _mm_bn_body(b_ref, g_ref, beta_ref, wt_ref, p_ref, o_ref):
    s = jnp.dot(wt_ref[...], p_ref[0] + p_ref[1],
                preferred_element_type=jnp.float32) + b_ref[...]
    o_ref[...] = _bn_relu(s, g_ref[...], beta_ref[...])


def _tc_mm_bn(p, b, g, beta, wt):
    return pl.pallas_call(
        _mm_bn_body,
        out_shape=jax.ShapeDtypeStruct((wt.shape[0], N), jnp.float32),
    )(b.reshape(-1, 1), g.reshape(-1, 1), beta.reshape(-1, 1), wt, p)


def _mm_bn_mm_body(b_ref, g_ref, beta_ref, wt_ref, w2t_ref, p_ref, o_ref):
    s = jnp.dot(wt_ref[...], p_ref[0] + p_ref[1],
                preferred_element_type=jnp.float32) + b_ref[...]
    h = _bn_relu(s, g_ref[...], beta_ref[...])
    o_ref[...] = jnp.dot(w2t_ref[...], h, preferred_element_type=jnp.float32)


def _tc_mm_bn_mm(p, b, g, beta, wt, w2t):
    return pl.pallas_call(
        _mm_bn_mm_body,
        out_shape=jax.ShapeDtypeStruct((w2t.shape[0], N), jnp.float32),
    )(b.reshape(-1, 1), g.reshape(-1, 1), beta.reshape(-1, 1), wt, w2t, p)


def _final_body(p_ref, w6t_ref, w7t_ref, b6_ref, b7_ref, zm_ref, zs_ref):
    q = p_ref[0] + p_ref[1]
    zm_ref[...] = (jnp.dot(w6t_ref[...], q,
                           preferred_element_type=jnp.float32)
                   + b6_ref[...]).T
    zs_ref[...] = (jnp.dot(w7t_ref[...], q,
                           preferred_element_type=jnp.float32)
                   + b7_ref[...]).T


def _tc_final(p, w6t, w7t, b6, b7):
    return pl.pallas_call(
        _final_body,
        out_shape=(jax.ShapeDtypeStruct((N, 32), jnp.float32),
                   jax.ShapeDtypeStruct((N, 32), jnp.float32)),
    )(p, w6t, w7t, b6.reshape(-1, 1), b7.reshape(-1, 1))


def kernel(x, edge_index, adj_values, W1, b1, W2, b2, W3, b3, W4, b4,
           W5, b5, W6, b6, W7, b7, g1, beta1, g2, beta2, g3, beta3,
           g4, beta4, g5, beta5):
    pad = E_PAD - E
    src = jnp.concatenate([edge_index[0], jnp.zeros((pad,), jnp.int32)])
    dst = jnp.concatenate([edge_index[1], jnp.zeros((pad,), jnp.int32)])
    adj = jnp.concatenate([adj_values, jnp.zeros((pad,), jnp.float32)])
    edata = jnp.stack(
        [src.reshape(NUM_CORES, CHUNKS, K_EDGES),
         dst.reshape(NUM_CORES, CHUNKS, K_EDGES),
         lax.bitcast_convert_type(adj, jnp.int32).reshape(
             NUM_CORES, CHUNKS, K_EDGES)], axis=2)
    edata = jnp.pad(edata, ((0, 0), (0, 1), (0, 0), (0, 0)))

    sup = _tc_mm0(x.T, W1.T)
    p = _spmm[16](sup, edata)
    h = _tc_sum_bn(p, b1, g1, beta1)
    q = _spmm[16](h, edata)
    h = _tc_mm_bn(q, b2, g2, beta2, W2.T)
    q = _spmm[32](h, edata)
    h = _tc_mm_bn(q, b3, g3, beta3, W3.T)
    q = _spmm[64](h, edata)
    sup = _tc_mm_bn_mm(q, b4, g4, beta4, W4.T, W5.T)
    p = _spmm[64](sup, edata)
    h = _tc_sum_bn(p, b5, g5, beta5)
    q = _spmm[64](h, edata)
    z_mean, z_std = _tc_final(q, W6.T, W7.T, b6, b7)
    return (z_mean, z_std)


# speedup vs baseline: 1.2614x; 1.0624x over previous
import functools

import jax
import jax.numpy as jnp
from jax import lax
from jax.experimental import pallas as pl
from jax.experimental.pallas import tpu as pltpu
from jax.experimental.pallas import tpu_sc as plsc

N = 10000
E = 320000

NUM_CORES = 2
NUM_SUBCORES = 16
K_EDGES = 512
EDGES_PER_CORE = 163840
E_PAD = EDGES_PER_CORE * NUM_CORES
CHUNKS = EDGES_PER_CORE // K_EDGES
GROUPS = K_EDGES // 16


def _make_spmm(do: int):
    C = do // NUM_SUBCORES
    mesh = plsc.VectorSubcoreMesh(
        core_axis_name="c", subcore_axis_name="s",
        num_cores=NUM_CORES, num_subcores=NUM_SUBCORES)

    @functools.partial(
        pl.kernel,
        out_type=jax.ShapeDtypeStruct((NUM_CORES, do, N), jnp.float32),
        mesh=mesh,
        compiler_params=pltpu.CompilerParams(
            needs_layout_passes=False, use_tc_tiling_on_sc=False),
        scratch_types=[
            pltpu.VMEM((2, K_EDGES), jnp.int32),
            pltpu.VMEM((2, K_EDGES), jnp.int32),
            pltpu.VMEM((C, N), jnp.float32),
            pltpu.VMEM((C, N), jnp.float32),
            pltpu.SemaphoreType.DMA,
            pltpu.SemaphoreType.DMA,
        ],
    )
    def spmm(sup_hbm, edata_hbm, out_hbm, eb0, eb1, sup, acc, se0, se1):
        cid = lax.axis_index("c")
        sid = lax.axis_index("s")
        f0 = sid * C

        pltpu.sync_copy(sup_hbm.at[pl.ds(f0, C)], sup)
        zero16 = jnp.zeros((16,), jnp.float32)

        def zfill(i, carry):
            for c in range(C):
                acc[c, pl.ds(i * 16, 16)] = zero16
            return carry

        lax.fori_loop(0, N // 16, zfill, 0)

        eb = (eb0, eb1)
        se = (se0, se1)

        def work(b, j):
            @plsc.parallel_loop(0, GROUPS, 1, unroll=8)
            def group(g):
                p16 = eb[b][0, pl.ds(g * 16, 16)]
                src16 = p16 >> 14
                dst16 = p16 & 16383
                a16 = plsc.bitcast(eb[b][1, pl.ds(g * 16, 16)], jnp.float32)
                for c in range(C):
                    v = plsc.load_gather(sup.at[c], [src16])
                    plsc.addupdate_scatter(acc.at[c], [dst16], v * a16)

        pltpu.async_copy(edata_hbm.at[cid, 0], eb0, se0)

        def pair(t, carry):
            pltpu.make_async_copy(edata_hbm.at[cid, 0], eb0, se0).wait()
            pltpu.async_copy(edata_hbm.at[cid, 2 * t + 1], eb1, se1)
            work(0, 2 * t)
            pltpu.make_async_copy(edata_hbm.at[cid, 0], eb1, se1).wait()
            pltpu.async_copy(edata_hbm.at[cid, 2 * t + 2], eb0, se0)
            work(1, 2 * t + 1)
            return carry

        lax.fori_loop(0, CHUNKS // 2, pair, 0)
        pltpu.make_async_copy(edata_hbm.at[cid, 0], eb0, se0).wait()

        pltpu.sync_copy(acc, out_hbm.at[cid, pl.ds(f0, C)])

    return spmm


_spmm = {d: _make_spmm(d) for d in (16, 32, 64)}


def _mm0_body(xt_ref, wt_ref, o_ref):
    o_ref[...] = jnp.dot(wt_ref[...], xt_ref[...],
                         preferred_element_type=jnp.float32)


def _tc_mm0(xt, wt):
    return pl.pallas_call(
        _mm0_body,
        out_shape=jax.ShapeDtypeStruct((wt.shape[0], N), jnp.float32),
    )(xt, wt)


def _bn_relu(s, g, beta):
    mu = jnp.mean(s, axis=1, keepdims=True)
    xc = s - mu
    var = jnp.mean(xc * xc, axis=1, keepdims=True)
    return jnp.maximum(xc * lax.rsqrt(var + 1e-5) * g + beta, 0.0)


def _sum_bn_body(b_ref, g_ref, beta_ref, p_ref, o_ref):
    s = p_ref[0] + p_ref[1] + b_ref[...]
    o_ref[...] = _bn_relu(s, g_ref[...], beta_ref[...])


def _tc_sum_bn(p, b, g, beta):
    return pl.pallas_call(
        _sum_bn_body,
        out_shape=jax.ShapeDtypeStruct((p.shape[1], N), jnp.float32),
    )(b.reshape(-1, 1), g.reshape(-1, 1), beta.reshape(-1, 1), p)


def _mm_bn_body(b_ref, g_ref, beta_ref, wt_ref, p_ref, o_ref):
    s = jnp.dot(wt_ref[...], p_ref[0] + p_ref[1],
                preferred_element_type=jnp.float32) + b_ref[...]
    o_ref[...] = _bn_relu(s, g_ref[...], beta_ref[...])


def _tc_mm_bn(p, b, g, beta, wt):
    return pl.pallas_call(
        _mm_bn_body,
        out_shape=jax.ShapeDtypeStruct((wt.shape[0], N), jnp.float32),
    )(b.reshape(-1, 1), g.reshape(-1, 1), beta.reshape(-1, 1), wt, p)


def _mm_bn_mm_body(b_ref, g_ref, beta_ref, wt_ref, w2t_ref, p_ref, o_ref):
    s = jnp.dot(wt_ref[...], p_ref[0] + p_ref[1],
                preferred_element_type=jnp.float32) + b_ref[...]
    h = _bn_relu(s, g_ref[...], beta_ref[...])
    o_ref[...] = jnp.dot(w2t_ref[...], h, preferred_element_type=jnp.float32)


def _tc_mm_bn_mm(p, b, g, beta, wt, w2t):
    return pl.pallas_call(
        _mm_bn_mm_body,
        out_shape=jax.ShapeDtypeStruct((w2t.shape[0], N), jnp.float32),
    )(b.reshape(-1, 1), g.reshape(-1, 1), beta.reshape(-1, 1), wt, w2t, p)


def _final_body(p_ref, w6t_ref, w7t_ref, b6_ref, b7_ref, zm_ref, zs_ref):
    q = p_ref[0] + p_ref[1]
    zm_ref[...] = (jnp.dot(w6t_ref[...], q,
                           preferred_element_type=jnp.float32)
                   + b6_ref[...]).T
    zs_ref[...] = (jnp.dot(w7t_ref[...], q,
                           preferred_element_type=jnp.float32)
                   + b7_ref[...]).T


def _tc_final(p, w6t, w7t, b6, b7):
    return pl.pallas_call(
        _final_body,
        out_shape=(jax.ShapeDtypeStruct((N, 32), jnp.float32),
                   jax.ShapeDtypeStruct((N, 32), jnp.float32)),
    )(p, w6t, w7t, b6.reshape(-1, 1), b7.reshape(-1, 1))


def kernel(x, edge_index, adj_values, W1, b1, W2, b2, W3, b3, W4, b4,
           W5, b5, W6, b6, W7, b7, g1, beta1, g2, beta2, g3, beta3,
           g4, beta4, g5, beta5):
    pad = E_PAD - E
    src = jnp.concatenate([edge_index[0], jnp.zeros((pad,), jnp.int32)])
    dst = jnp.concatenate([edge_index[1], jnp.zeros((pad,), jnp.int32)])
    adj = jnp.concatenate([adj_values, jnp.zeros((pad,), jnp.float32)])
    packed = (src << 14) | dst
    edata = jnp.stack(
        [packed.reshape(NUM_CORES, CHUNKS, K_EDGES),
         lax.bitcast_convert_type(adj, jnp.int32).reshape(
             NUM_CORES, CHUNKS, K_EDGES)], axis=2)
    edata = jnp.pad(edata, ((0, 0), (0, 1), (0, 0), (0, 0)))

    sup = _tc_mm0(x.T, W1.T)
    p = _spmm[16](sup, edata)
    h = _tc_sum_bn(p, b1, g1, beta1)
    q = _spmm[16](h, edata)
    h = _tc_mm_bn(q, b2, g2, beta2, W2.T)
    q = _spmm[32](h, edata)
    h = _tc_mm_bn(q, b3, g3, beta3, W3.T)
    q = _spmm[64](h, edata)
    sup = _tc_mm_bn_mm(q, b4, g4, beta4, W4.T, W5.T)
    p = _spmm[64](sup, edata)
    h = _tc_sum_bn(p, b5, g5, beta5)
    q = _spmm[64](h, edata)
    z_mean, z_std = _tc_final(q, W6.T, W7.T, b6, b7)
    return (z_mean, z_std)


# speedup vs baseline: 2.2087x; 1.7510x over previous
import functools

import jax
import jax.numpy as jnp
from jax import lax
from jax.experimental import pallas as pl
from jax.experimental.pallas import tpu as pltpu
from jax.experimental.pallas import tpu_sc as plsc

N = 10000
E = 320000

NUM_CORES = 2
NUM_SUBCORES = 16
K_EDGES = 2048
EDGES_PER_CORE = 163840
E_PAD = EDGES_PER_CORE * NUM_CORES
CHUNKS = EDGES_PER_CORE // K_EDGES
GROUPS = K_EDGES // 16


def _make_spmm(do: int):
    C = do // NUM_SUBCORES
    mesh = plsc.VectorSubcoreMesh(
        core_axis_name="c", subcore_axis_name="s",
        num_cores=NUM_CORES, num_subcores=NUM_SUBCORES)

    @functools.partial(
        pl.kernel,
        out_type=jax.ShapeDtypeStruct((NUM_CORES, do, N), jnp.float32),
        mesh=mesh,
        compiler_params=pltpu.CompilerParams(
            needs_layout_passes=False, use_tc_tiling_on_sc=False),
        scratch_types=[
            pltpu.VMEM((2, K_EDGES), jnp.int32),
            pltpu.VMEM((2, K_EDGES), jnp.int32),
            pltpu.VMEM((C, N), jnp.float32),
            pltpu.VMEM((C, N), jnp.float32),
            pltpu.SemaphoreType.DMA,
            pltpu.SemaphoreType.DMA,
        ],
    )
    def spmm(sup_hbm, edata_hbm, out_hbm, eb0, eb1, sup, acc, se0, se1):
        cid = lax.axis_index("c")
        sid = lax.axis_index("s")
        f0 = sid * C

        pltpu.sync_copy(sup_hbm.at[pl.ds(f0, C)], sup)
        zero16 = jnp.zeros((16,), jnp.float32)

        def zfill(i, carry):
            for c in range(C):
                acc[c, pl.ds(i * 16, 16)] = zero16
            return carry

        lax.fori_loop(0, N // 16, zfill, 0)

        eb = (eb0, eb1)
        se = (se0, se1)

        def work(b, j):
            @plsc.parallel_loop(0, GROUPS, 1, unroll=8)
            def group(g):
                p16 = eb[b][0, pl.ds(g * 16, 16)]
                src16 = p16 >> 14
                dst16 = p16 & 16383
                a16 = plsc.bitcast(eb[b][1, pl.ds(g * 16, 16)], jnp.float32)
                for c in range(C):
                    v = plsc.load_gather(sup.at[c], [src16])
                    plsc.addupdate_scatter(acc.at[c], [dst16], v * a16)

        pltpu.async_copy(edata_hbm.at[cid, 0], eb0, se0)

        def pair(t, carry):
            pltpu.make_async_copy(edata_hbm.at[cid, 0], eb0, se0).wait()
            pltpu.async_copy(edata_hbm.at[cid, 2 * t + 1], eb1, se1)
            work(0, 2 * t)
            pltpu.make_async_copy(edata_hbm.at[cid, 0], eb1, se1).wait()
            pltpu.async_copy(edata_hbm.at[cid, 2 * t + 2], eb0, se0)
            work(1, 2 * t + 1)
            return carry

        lax.fori_loop(0, CHUNKS // 2, pair, 0)
        pltpu.make_async_copy(edata_hbm.at[cid, 0], eb0, se0).wait()

        pltpu.sync_copy(acc, out_hbm.at[cid, pl.ds(f0, C)])

    return spmm


_spmm = {d: _make_spmm(d) for d in (16, 32, 64)}


def _mm0_body(xt_ref, wt_ref, o_ref):
    o_ref[...] = jnp.dot(wt_ref[...], xt_ref[...],
                         preferred_element_type=jnp.float32)


def _tc_mm0(xt, wt):
    return pl.pallas_call(
        _mm0_body,
        out_shape=jax.ShapeDtypeStruct((wt.shape[0], N), jnp.float32),
    )(xt, wt)


def _bn_relu(s, g, beta):
    mu = jnp.mean(s, axis=1, keepdims=True)
    xc = s - mu
    var = jnp.mean(xc * xc, axis=1, keepdims=True)
    return jnp.maximum(xc * lax.rsqrt(var + 1e-5) * g + beta, 0.0)


def _sum_bn_body(b_ref, g_ref, beta_ref, p_ref, o_ref):
    s = p_ref[0] + p_ref[1] + b_ref[...]
    o_ref[...] = _bn_relu(s, g_ref[...], beta_ref[...])


def _tc_sum_bn(p, b, g, beta):
    return pl.pallas_call(
        _sum_bn_body,
        out_shape=jax.ShapeDtypeStruct((p.shape[1], N), jnp.float32),
    )(b.reshape(-1, 1), g.reshape(-1, 1), beta.reshape(-1, 1), p)


def _mm_bn_body(b_ref, g_ref, beta_ref, wt_ref, p_ref, o_ref):
    s = jnp.dot(wt_ref[...], p_ref[0] + p_ref[1],
                preferred_element_type=jnp.float32) + b_ref[...]
    o_ref[...] = _bn_relu(s, g_ref[...], beta_ref[...])


def _tc_mm_bn(p, b, g, beta, wt):
    return pl.pallas_call(
        _mm_bn_body,
        out_shape=jax.ShapeDtypeStruct((wt.shape[0], N), jnp.float32),
    )(b.reshape(-1, 1), g.reshape(-1, 1), beta.reshape(-1, 1), wt, p)


def _mm_bn_mm_body(b_ref, g_ref, beta_ref, wt_ref, w2t_ref, p_ref, o_ref):
    s = jnp.dot(wt_ref[...], p_ref[0] + p_ref[1],
                preferred_element_type=jnp.float32) + b_ref[...]
    h = _bn_relu(s, g_ref[...], beta_ref[...])
    o_ref[...] = jnp.dot(w2t_ref[...], h, preferred_element_type=jnp.float32)


def _tc_mm_bn_mm(p, b, g, beta, wt, w2t):
    return pl.pallas_call(
        _mm_bn_mm_body,
        out_shape=jax.ShapeDtypeStruct((w2t.shape[0], N), jnp.float32),
    )(b.reshape(-1, 1), g.reshape(-1, 1), beta.reshape(-1, 1), wt, w2t, p)


def _final_body(p_ref, w6t_ref, w7t_ref, b6_ref, b7_ref, zm_ref, zs_ref):
    q = p_ref[0] + p_ref[1]
    zm_ref[...] = (jnp.dot(w6t_ref[...], q,
                           preferred_element_type=jnp.float32)
                   + b6_ref[...]).T
    zs_ref[...] = (jnp.dot(w7t_ref[...], q,
                           preferred_element_type=jnp.float32)
                   + b7_ref[...]).T


def _tc_final(p, w6t, w7t, b6, b7):
    return pl.pallas_call(
        _final_body,
        out_shape=(jax.ShapeDtypeStruct((N, 32), jnp.float32),
                   jax.ShapeDtypeStruct((N, 32), jnp.float32)),
    )(p, w6t, w7t, b6.reshape(-1, 1), b7.reshape(-1, 1))


def kernel(x, edge_index, adj_values, W1, b1, W2, b2, W3, b3, W4, b4,
           W5, b5, W6, b6, W7, b7, g1, beta1, g2, beta2, g3, beta3,
           g4, beta4, g5, beta5):
    pad = E_PAD - E
    src = jnp.concatenate([edge_index[0], jnp.zeros((pad,), jnp.int32)])
    dst = jnp.concatenate([edge_index[1], jnp.zeros((pad,), jnp.int32)])
    adj = jnp.concatenate([adj_values, jnp.zeros((pad,), jnp.float32)])
    packed = (src << 14) | dst
    edata = jnp.stack(
        [packed.reshape(NUM_CORES, CHUNKS, K_EDGES),
         lax.bitcast_convert_type(adj, jnp.int32).reshape(
             NUM_CORES, CHUNKS, K_EDGES)], axis=2)
    edata = jnp.pad(edata, ((0, 0), (0, 1), (0, 0), (0, 0)))

    sup = _tc_mm0(x.T, W1.T)
    p = _spmm[16](sup, edata)
    h = _tc_sum_bn(p, b1, g1, beta1)
    q = _spmm[16](h, edata)
    h = _tc_mm_bn(q, b2, g2, beta2, W2.T)
    q = _spmm[32](h, edata)
    h = _tc_mm_bn(q, b3, g3, beta3, W3.T)
    q = _spmm[64](h, edata)
    sup = _tc_mm_bn_mm(q, b4, g4, beta4, W4.T, W5.T)
    p = _spmm[64](sup, edata)
    h = _tc_sum_bn(p, b5, g5, beta5)
    q = _spmm[64](h, edata)
    z_mean, z_std = _tc_final(q, W6.T, W7.T, b6, b7)
    return (z_mean, z_std)


# speedup vs baseline: 2.3782x; 1.0767x over previous
import functools

import jax
import jax.numpy as jnp
from jax import lax
from jax.experimental import pallas as pl
from jax.experimental.pallas import tpu as pltpu
from jax.experimental.pallas import tpu_sc as plsc

N = 10000
E = 320000

NUM_CORES = 2
NUM_SUBCORES = 16
K_EDGES = 8192
EDGES_PER_CORE = 163840
E_PAD = EDGES_PER_CORE * NUM_CORES
CHUNKS = EDGES_PER_CORE // K_EDGES
GROUPS = K_EDGES // 16


def _make_spmm(do: int):
    C = do // NUM_SUBCORES
    mesh = plsc.VectorSubcoreMesh(
        core_axis_name="c", subcore_axis_name="s",
        num_cores=NUM_CORES, num_subcores=NUM_SUBCORES)

    @functools.partial(
        pl.kernel,
        out_type=jax.ShapeDtypeStruct((NUM_CORES, do, N), jnp.float32),
        mesh=mesh,
        compiler_params=pltpu.CompilerParams(
            needs_layout_passes=False, use_tc_tiling_on_sc=False),
        scratch_types=[
            pltpu.VMEM((2, K_EDGES), jnp.int32),
            pltpu.VMEM((2, K_EDGES), jnp.int32),
            pltpu.VMEM((C, N), jnp.float32),
            pltpu.VMEM((C, N), jnp.float32),
            pltpu.SemaphoreType.DMA,
            pltpu.SemaphoreType.DMA,
        ],
    )
    def spmm(sup_hbm, edata_hbm, out_hbm, eb0, eb1, sup, acc, se0, se1):
        cid = lax.axis_index("c")
        sid = lax.axis_index("s")
        f0 = sid * C

        pltpu.sync_copy(sup_hbm.at[pl.ds(f0, C)], sup)
        zero16 = jnp.zeros((16,), jnp.float32)

        def zfill(i, carry):
            for c in range(C):
                acc[c, pl.ds(i * 16, 16)] = zero16
            return carry

        lax.fori_loop(0, N // 16, zfill, 0)

        eb = (eb0, eb1)
        se = (se0, se1)

        def work(b, j):
            @plsc.parallel_loop(0, GROUPS, 1, unroll=8)
            def group(g):
                p16 = eb[b][0, pl.ds(g * 16, 16)]
                src16 = p16 >> 14
                dst16 = p16 & 16383
                a16 = plsc.bitcast(eb[b][1, pl.ds(g * 16, 16)], jnp.float32)
                for c in range(C):
                    v = plsc.load_gather(sup.at[c], [src16])
                    plsc.addupdate_scatter(acc.at[c], [dst16], v * a16)

        pltpu.async_copy(edata_hbm.at[cid, 0], eb0, se0)

        def pair(t, carry):
            pltpu.make_async_copy(edata_hbm.at[cid, 0], eb0, se0).wait()
            pltpu.async_copy(edata_hbm.at[cid, 2 * t + 1], eb1, se1)
            work(0, 2 * t)
            pltpu.make_async_copy(edata_hbm.at[cid, 0], eb1, se1).wait()
            pltpu.async_copy(edata_hbm.at[cid, 2 * t + 2], eb0, se0)
            work(1, 2 * t + 1)
            return carry

        lax.fori_loop(0, CHUNKS // 2, pair, 0)
        pltpu.make_async_copy(edata_hbm.at[cid, 0], eb0, se0).wait()

        pltpu.sync_copy(acc, out_hbm.at[cid, pl.ds(f0, C)])

    return spmm


_spmm = {d: _make_spmm(d) for d in (16, 32, 64)}


def _mm0_body(xt_ref, wt_ref, o_ref):
    o_ref[...] = jnp.dot(wt_ref[...], xt_ref[...],
                         preferred_element_type=jnp.float32)


def _tc_mm0(xt, wt):
    return pl.pallas_call(
        _mm0_body,
        out_shape=jax.ShapeDtypeStruct((wt.shape[0], N), jnp.float32),
    )(xt, wt)


def _bn_relu(s, g, beta):
    mu = jnp.mean(s, axis=1, keepdims=True)
    xc = s - mu
    var = jnp.mean(xc * xc, axis=1, keepdims=True)
    return jnp.maximum(xc * lax.rsqrt(var + 1e-5) * g + beta, 0.0)


def _sum_bn_body(b_ref, g_ref, beta_ref, p_ref, o_ref):
    s = p_ref[0] + p_ref[1] + b_ref[...]
    o_ref[...] = _bn_relu(s, g_ref[...], beta_ref[...])


def _tc_sum_bn(p, b, g, beta):
    return pl.pallas_call(
        _sum_bn_body,
        out_shape=jax.ShapeDtypeStruct((p.shape[1], N), jnp.float32),
    )(b.reshape(-1, 1), g.reshape(-1, 1), beta.reshape(-1, 1), p)


def _mm_bn_body(b_ref, g_ref, beta_ref, wt_ref, p_ref, o_ref):
    s = jnp.dot(wt_ref[...], p_ref[0] + p_ref[1],
                preferred_element_type=jnp.float32) + b_ref[...]
    o_ref[...] = _bn_relu(s, g_ref[...], beta_ref[...])


def _tc_mm_bn(p, b, g, beta, wt):
    return pl.pallas_call(
        _mm_bn_body,
        out_shape=jax.ShapeDtypeStruct((wt.shape[0], N), jnp.float32),
    )(b.reshape(-1, 1), g.reshape(-1, 1), beta.reshape(-1, 1), wt, p)


def _mm_bn_mm_body(b_ref, g_ref, beta_ref, wt_ref, w2t_ref, p_ref, o_ref):
    s = jnp.dot(wt_ref[...], p_ref[0] + p_ref[1],
                preferred_element_type=jnp.float32) + b_ref[...]
    h = _bn_relu(s, g_ref[...], beta_ref[...])
    o_ref[...] = jnp.dot(w2t_ref[...], h, preferred_element_type=jnp.float32)


def _tc_mm_bn_mm(p, b, g, beta, wt, w2t):
    return pl.pallas_call(
        _mm_bn_mm_body,
        out_shape=jax.ShapeDtypeStruct((w2t.shape[0], N), jnp.float32),
    )(b.reshape(-1, 1), g.reshape(-1, 1), beta.reshape(-1, 1), wt, w2t, p)


def _final_body(p_ref, w6t_ref, w7t_ref, b6_ref, b7_ref, zm_ref, zs_ref):
    q = p_ref[0] + p_ref[1]
    zm_ref[...] = (jnp.dot(w6t_ref[...], q,
                           preferred_element_type=jnp.float32)
                   + b6_ref[...]).T
    zs_ref[...] = (jnp.dot(w7t_ref[...], q,
                           preferred_element_type=jnp.float32)
                   + b7_ref[...]).T


def _tc_final(p, w6t, w7t, b6, b7):
    return pl.pallas_call(
        _final_body,
        out_shape=(jax.ShapeDtypeStruct((N, 32), jnp.float32),
                   jax.ShapeDtypeStruct((N, 32), jnp.float32)),
    )(p, w6t, w7t, b6.reshape(-1, 1), b7.reshape(-1, 1))


def kernel(x, edge_index, adj_values, W1, b1, W2, b2, W3, b3, W4, b4,
           W5, b5, W6, b6, W7, b7, g1, beta1, g2, beta2, g3, beta3,
           g4, beta4, g5, beta5):
    pad = E_PAD - E
    src = jnp.concatenate([edge_index[0], jnp.zeros((pad,), jnp.int32)])
    dst = jnp.concatenate([edge_index[1], jnp.zeros((pad,), jnp.int32)])
    adj = jnp.concatenate([adj_values, jnp.zeros((pad,), jnp.float32)])
    packed = (src << 14) | dst
    edata = jnp.stack(
        [packed.reshape(NUM_CORES, CHUNKS, K_EDGES),
         lax.bitcast_convert_type(adj, jnp.int32).reshape(
             NUM_CORES, CHUNKS, K_EDGES)], axis=2)
    edata = jnp.pad(edata, ((0, 0), (0, 1), (0, 0), (0, 0)))

    sup = _tc_mm0(x.T, W1.T)
    p = _spmm[16](sup, edata)
    h = _tc_sum_bn(p, b1, g1, beta1)
    q = _spmm[16](h, edata)
    h = _tc_mm_bn(q, b2, g2, beta2, W2.T)
    q = _spmm[32](h, edata)
    h = _tc_mm_bn(q, b3, g3, beta3, W3.T)
    q = _spmm[64](h, edata)
    sup = _tc_mm_bn_mm(q, b4, g4, beta4, W4.T, W5.T)
    p = _spmm[64](sup, edata)
    h = _tc_sum_bn(p, b5, g5, beta5)
    q = _spmm[64](h, edata)
    z_mean, z_std = _tc_final(q, W6.T, W7.T, b6, b7)
    return (z_mean, z_std)


# speedup vs baseline: 2.3947x; 1.0069x over previous
import functools

import jax
import jax.numpy as jnp
from jax import lax
from jax.experimental import pallas as pl
from jax.experimental.pallas import tpu as pltpu
from jax.experimental.pallas import tpu_sc as plsc

N = 10000
E = 320000

NUM_CORES = 2
NUM_SUBCORES = 16
K_EDGES = 8192
EDGES_PER_CORE = 163840
E_PAD = EDGES_PER_CORE * NUM_CORES
CHUNKS = EDGES_PER_CORE // K_EDGES
GROUPS = K_EDGES // 16


def _make_spmm(do: int):
    C = do // NUM_SUBCORES
    mesh = plsc.VectorSubcoreMesh(
        core_axis_name="c", subcore_axis_name="s",
        num_cores=NUM_CORES, num_subcores=NUM_SUBCORES)

    @functools.partial(
        pl.kernel,
        out_type=jax.ShapeDtypeStruct((NUM_CORES, do, N), jnp.float32),
        mesh=mesh,
        compiler_params=pltpu.CompilerParams(
            needs_layout_passes=False, use_tc_tiling_on_sc=False),
        scratch_types=[
            pltpu.VMEM((2, K_EDGES), jnp.int32),
            pltpu.VMEM((2, K_EDGES), jnp.int32),
            pltpu.VMEM((C, N), jnp.float32),
            pltpu.VMEM((C, N), jnp.float32),
            pltpu.SemaphoreType.DMA,
            pltpu.SemaphoreType.DMA,
        ],
    )
    def spmm(sup_hbm, edata_hbm, out_hbm, eb0, eb1, sup, acc, se0, se1):
        cid = lax.axis_index("c")
        sid = lax.axis_index("s")
        f0 = sid * C

        pltpu.sync_copy(sup_hbm.at[pl.ds(f0, C)], sup)
        zero16 = jnp.zeros((16,), jnp.float32)

        def zfill(i, carry):
            for c in range(C):
                acc[c, pl.ds(i * 16, 16)] = zero16
            return carry

        lax.fori_loop(0, N // 16, zfill, 0)

        eb = (eb0, eb1)
        se = (se0, se1)

        def work(b, j):
            @plsc.parallel_loop(0, GROUPS, 1, unroll=16)
            def group(g):
                p16 = eb[b][0, pl.ds(g * 16, 16)]
                src16 = p16 >> 14
                dst16 = p16 & 16383
                a16 = plsc.bitcast(eb[b][1, pl.ds(g * 16, 16)], jnp.float32)
                for c in range(C):
                    v = plsc.load_gather(sup.at[c], [src16])
                    plsc.addupdate_scatter(acc.at[c], [dst16], v * a16)

        pltpu.async_copy(edata_hbm.at[cid, 0], eb0, se0)

        def pair(t, carry):
            pltpu.make_async_copy(edata_hbm.at[cid, 0], eb0, se0).wait()
            pltpu.async_copy(edata_hbm.at[cid, 2 * t + 1], eb1, se1)
            work(0, 2 * t)
            pltpu.make_async_copy(edata_hbm.at[cid, 0], eb1, se1).wait()
            pltpu.async_copy(edata_hbm.at[cid, 2 * t + 2], eb0, se0)
            work(1, 2 * t + 1)
            return carry

        lax.fori_loop(0, CHUNKS // 2, pair, 0)
        pltpu.make_async_copy(edata_hbm.at[cid, 0], eb0, se0).wait()

        pltpu.sync_copy(acc, out_hbm.at[cid, pl.ds(f0, C)])

    return spmm


_spmm = {d: _make_spmm(d) for d in (16, 32, 64)}


def _mm0_body(xt_ref, wt_ref, o_ref):
    o_ref[...] = jnp.dot(wt_ref[...], xt_ref[...],
                         preferred_element_type=jnp.float32)


def _tc_mm0(xt, wt):
    return pl.pallas_call(
        _mm0_body,
        out_shape=jax.ShapeDtypeStruct((wt.shape[0], N), jnp.float32),
    )(xt, wt)


def _bn_relu(s, g, beta):
    mu = jnp.mean(s, axis=1, keepdims=True)
    xc = s - mu
    var = jnp.mean(xc * xc, axis=1, keepdims=True)
    return jnp.maximum(xc * lax.rsqrt(var + 1e-5) * g + beta, 0.0)


def _sum_bn_body(b_ref, g_ref, beta_ref, p_ref, o_ref):
    s = p_ref[0] + p_ref[1] + b_ref[...]
    o_ref[...] = _bn_relu(s, g_ref[...], beta_ref[...])


def _tc_sum_bn(p, b, g, beta):
    return pl.pallas_call(
        _sum_bn_body,
        out_shape=jax.ShapeDtypeStruct((p.shape[1], N), jnp.float32),
    )(b.reshape(-1, 1), g.reshape(-1, 1), beta.reshape(-1, 1), p)


def _mm_bn_body(b_ref, g_ref, beta_ref, wt_ref, p_ref, o_ref):
    s = jnp.dot(wt_ref[...], p_ref[0] + p_ref[1],
                preferred_element_type=jnp.float32) + b_ref[...]
    o_ref[...] = _bn_relu(s, g_ref[...], beta_ref[...])


def _tc_mm_bn(p, b, g, beta, wt):
    return pl.pallas_call(
        _mm_bn_body,
        out_shape=jax.ShapeDtypeStruct((wt.shape[0], N), jnp.float32),
    )(b.reshape(-1, 1), g.reshape(-1, 1), beta.reshape(-1, 1), wt, p)


def _mm_bn_mm_body(b_ref, g_ref, beta_ref, wt_ref, w2t_ref, p_ref, o_ref):
    s = jnp.dot(wt_ref[...], p_ref[0] + p_ref[1],
                preferred_element_type=jnp.float32) + b_ref[...]
    h = _bn_relu(s, g_ref[...], beta_ref[...])
    o_ref[...] = jnp.dot(w2t_ref[...], h, preferred_element_type=jnp.float32)


def _tc_mm_bn_mm(p, b, g, beta, wt, w2t):
    return pl.pallas_call(
        _mm_bn_mm_body,
        out_shape=jax.ShapeDtypeStruct((w2t.shape[0], N), jnp.float32),
    )(b.reshape(-1, 1), g.reshape(-1, 1), beta.reshape(-1, 1), wt, w2t, p)


def _final_body(p_ref, w6t_ref, w7t_ref, b6_ref, b7_ref, zm_ref, zs_ref):
    q = p_ref[0] + p_ref[1]
    zm_ref[...] = (jnp.dot(w6t_ref[...], q,
                           preferred_element_type=jnp.float32)
                   + b6_ref[...]).T
    zs_ref[...] = (jnp.dot(w7t_ref[...], q,
                           preferred_element_type=jnp.float32)
                   + b7_ref[...]).T


def _tc_final(p, w6t, w7t, b6, b7):
    return pl.pallas_call(
        _final_body,
        out_shape=(jax.ShapeDtypeStruct((N, 32), jnp.float32),
                   jax.ShapeDtypeStruct((N, 32), jnp.float32)),
    )(p, w6t, w7t, b6.reshape(-1, 1), b7.reshape(-1, 1))


def kernel(x, edge_index, adj_values, W1, b1, W2, b2, W3, b3, W4, b4,
           W5, b5, W6, b6, W7, b7, g1, beta1, g2, beta2, g3, beta3,
           g4, beta4, g5, beta5):
    pad = E_PAD - E
    src = jnp.concatenate([edge_index[0], jnp.zeros((pad,), jnp.int32)])
    dst = jnp.concatenate([edge_index[1], jnp.zeros((pad,), jnp.int32)])
    adj = jnp.concatenate([adj_values, jnp.zeros((pad,), jnp.float32)])
    packed = (src << 14) | dst
    edata = jnp.stack(
        [packed.reshape(NUM_CORES, CHUNKS, K_EDGES),
         lax.bitcast_convert_type(adj, jnp.int32).reshape(
             NUM_CORES, CHUNKS, K_EDGES)], axis=2)
    edata = jnp.pad(edata, ((0, 0), (0, 1), (0, 0), (0, 0)))

    sup = _tc_mm0(x.T, W1.T)
    p = _spmm[16](sup, edata)
    h = _tc_sum_bn(p, b1, g1, beta1)
    q = _spmm[16](h, edata)
    h = _tc_mm_bn(q, b2, g2, beta2, W2.T)
    q = _spmm[32](h, edata)
    h = _tc_mm_bn(q, b3, g3, beta3, W3.T)
    q = _spmm[64](h, edata)
    sup = _tc_mm_bn_mm(q, b4, g4, beta4, W4.T, W5.T)
    p = _spmm[64](sup, edata)
    h = _tc_sum_bn(p, b5, g5, beta5)
    q = _spmm[64](h, edata)
    z_mean, z_std = _tc_final(q, W6.T, W7.T, b6, b7)
    return (z_mean, z_std)


# speedup vs baseline: 2.4147x; 1.0083x over previous
import functools

import jax
import jax.numpy as jnp
from jax import lax
from jax.experimental import pallas as pl
from jax.experimental.pallas import tpu as pltpu
from jax.experimental.pallas import tpu_sc as plsc

N = 10000
E = 320000

NUM_CORES = 2
NUM_SUBCORES = 16
K_EDGES = 8192
EDGES_PER_CORE = 163840
E_PAD = EDGES_PER_CORE * NUM_CORES
CHUNKS = EDGES_PER_CORE // K_EDGES
GROUPS = K_EDGES // 16


def _make_spmm(do: int):
    C = do // NUM_SUBCORES
    mesh = plsc.VectorSubcoreMesh(
        core_axis_name="c", subcore_axis_name="s",
        num_cores=NUM_CORES, num_subcores=NUM_SUBCORES)

    @functools.partial(
        pl.kernel,
        out_type=jax.ShapeDtypeStruct((NUM_CORES, do, N), jnp.float32),
        mesh=mesh,
        compiler_params=pltpu.CompilerParams(
            needs_layout_passes=False, use_tc_tiling_on_sc=False),
        scratch_types=[
            pltpu.VMEM((2, K_EDGES), jnp.int32),
            pltpu.VMEM((2, K_EDGES), jnp.int32),
            pltpu.VMEM((C, N), jnp.float32),
            pltpu.VMEM((C, N), jnp.float32),
            pltpu.SemaphoreType.DMA,
            pltpu.SemaphoreType.DMA,
        ],
    )
    def spmm(sup_hbm, edata_hbm, out_hbm, eb0, eb1, sup, acc, se0, se1):
        cid = lax.axis_index("c")
        sid = lax.axis_index("s")
        f0 = sid * C

        pltpu.sync_copy(sup_hbm.at[pl.ds(f0, C)], sup)
        zero16 = jnp.zeros((16,), jnp.float32)

        def zfill(i, carry):
            for c in range(C):
                acc[c, pl.ds(i * 16, 16)] = zero16
            return carry

        lax.fori_loop(0, N // 16, zfill, 0)

        eb = (eb0, eb1)
        se = (se0, se1)

        def work(b, j):
            @plsc.parallel_loop(0, GROUPS, 1, unroll=32)
            def group(g):
                p16 = eb[b][0, pl.ds(g * 16, 16)]
                src16 = p16 >> 14
                dst16 = p16 & 16383
                a16 = plsc.bitcast(eb[b][1, pl.ds(g * 16, 16)], jnp.float32)
                for c in range(C):
                    v = plsc.load_gather(sup.at[c], [src16])
                    plsc.addupdate_scatter(acc.at[c], [dst16], v * a16)

        pltpu.async_copy(edata_hbm.at[cid, 0], eb0, se0)

        def pair(t, carry):
            pltpu.make_async_copy(edata_hbm.at[cid, 0], eb0, se0).wait()
            pltpu.async_copy(edata_hbm.at[cid, 2 * t + 1], eb1, se1)
            work(0, 2 * t)
            pltpu.make_async_copy(edata_hbm.at[cid, 0], eb1, se1).wait()
            pltpu.async_copy(edata_hbm.at[cid, 2 * t + 2], eb0, se0)
            work(1, 2 * t + 1)
            return carry

        lax.fori_loop(0, CHUNKS // 2, pair, 0)
        pltpu.make_async_copy(edata_hbm.at[cid, 0], eb0, se0).wait()

        pltpu.sync_copy(acc, out_hbm.at[cid, pl.ds(f0, C)])

    return spmm


_spmm = {d: _make_spmm(d) for d in (16, 32, 64)}


def _mm0_body(xt_ref, wt_ref, o_ref):
    o_ref[...] = jnp.dot(wt_ref[...], xt_ref[...],
                         preferred_element_type=jnp.float32)


def _tc_mm0(xt, wt):
    return pl.pallas_call(
        _mm0_body,
        out_shape=jax.ShapeDtypeStruct((wt.shape[0], N), jnp.float32),
    )(xt, wt)


def _bn_relu(s, g, beta):
    mu = jnp.mean(s, axis=1, keepdims=True)
    xc = s - mu
    var = jnp.mean(xc * xc, axis=1, keepdims=True)
    return jnp.maximum(xc * lax.rsqrt(var + 1e-5) * g + beta, 0.0)


def _sum_bn_body(b_ref, g_ref, beta_ref, p_ref, o_ref):
    s = p_ref[0] + p_ref[1] + b_ref[...]
    o_ref[...] = _bn_relu(s, g_ref[...], beta_ref[...])


def _tc_sum_bn(p, b, g, beta):
    return pl.pallas_call(
        _sum_bn_body,
        out_shape=jax.ShapeDtypeStruct((p.shape[1], N), jnp.float32),
    )(b.reshape(-1, 1), g.reshape(-1, 1), beta.reshape(-1, 1), p)


def _mm_bn_body(b_ref, g_ref, beta_ref, wt_ref, p_ref, o_ref):
    s = jnp.dot(wt_ref[...], p_ref[0] + p_ref[1],
                preferred_element_type=jnp.float32) + b_ref[...]
    o_ref[...] = _bn_relu(s, g_ref[...], beta_ref[...])


def _tc_mm_bn(p, b, g, beta, wt):
    return pl.pallas_call(
        _mm_bn_body,
        out_shape=jax.ShapeDtypeStruct((wt.shape[0], N), jnp.float32),
    )(b.reshape(-1, 1), g.reshape(-1, 1), beta.reshape(-1, 1), wt, p)


def _mm_bn_mm_body(b_ref, g_ref, beta_ref, wt_ref, w2t_ref, p_ref, o_ref):
    s = jnp.dot(wt_ref[...], p_ref[0] + p_ref[1],
                preferred_element_type=jnp.float32) + b_ref[...]
    h = _bn_relu(s, g_ref[...], beta_ref[...])
    o_ref[...] = jnp.dot(w2t_ref[...], h, preferred_element_type=jnp.float32)


def _tc_mm_bn_mm(p, b, g, beta, wt, w2t):
    return pl.pallas_call(
        _mm_bn_mm_body,
        out_shape=jax.ShapeDtypeStruct((w2t.shape[0], N), jnp.float32),
    )(b.reshape(-1, 1), g.reshape(-1, 1), beta.reshape(-1, 1), wt, w2t, p)


def _final_body(p_ref, w6t_ref, w7t_ref, b6_ref, b7_ref, zm_ref, zs_ref):
    q = p_ref[0] + p_ref[1]
    zm_ref[...] = (jnp.dot(w6t_ref[...], q,
                           preferred_element_type=jnp.float32)
                   + b6_ref[...]).T
    zs_ref[...] = (jnp.dot(w7t_ref[...], q,
                           preferred_element_type=jnp.float32)
                   + b7_ref[...]).T


def _tc_final(p, w6t, w7t, b6, b7):
    return pl.pallas_call(
        _final_body,
        out_shape=(jax.ShapeDtypeStruct((N, 32), jnp.float32),
                   jax.ShapeDtypeStruct((N, 32), jnp.float32)),
    )(p, w6t, w7t, b6.reshape(-1, 1), b7.reshape(-1, 1))


def kernel(x, edge_index, adj_values, W1, b1, W2, b2, W3, b3, W4, b4,
           W5, b5, W6, b6, W7, b7, g1, beta1, g2, beta2, g3, beta3,
           g4, beta4, g5, beta5):
    pad = E_PAD - E
    src = jnp.concatenate([edge_index[0], jnp.zeros((pad,), jnp.int32)])
    dst = jnp.concatenate([edge_index[1], jnp.zeros((pad,), jnp.int32)])
    adj = jnp.concatenate([adj_values, jnp.zeros((pad,), jnp.float32)])
    packed = (src << 14) | dst
    edata = jnp.stack(
        [packed.reshape(NUM_CORES, CHUNKS, K_EDGES),
         lax.bitcast_convert_type(adj, jnp.int32).reshape(
             NUM_CORES, CHUNKS, K_EDGES)], axis=2)
    edata = jnp.pad(edata, ((0, 0), (0, 1), (0, 0), (0, 0)))

    sup = _tc_mm0(x.T, W1.T)
    p = _spmm[16](sup, edata)
    h = _tc_sum_bn(p, b1, g1, beta1)
    q = _spmm[16](h, edata)
    h = _tc_mm_bn(q, b2, g2, beta2, W2.T)
    q = _spmm[32](h, edata)
    h = _tc_mm_bn(q, b3, g3, beta3, W3.T)
    q = _spmm[64](h, edata)
    sup = _tc_mm_bn_mm(q, b4, g4, beta4, W4.T, W5.T)
    p = _spmm[64](sup, edata)
    h = _tc_sum_bn(p, b5, g5, beta5)
    q = _spmm[64](h, edata)
    z_mean, z_std = _tc_final(q, W6.T, W7.T, b6, b7)
    return (z_mean, z_std)
